# Initial kernel scaffold; baseline (speedup 1.0000x reference)
import jax
import jax.numpy as jnp
from jax.experimental import pallas as pl


def kernel(f_oo_vis, spatial_branch_output, graphical_branch_output, obj_pairs, num_rels, W1, b1, W2, b2, W3, b3):
    raise NotImplementedError("write your pallas kernel here")


# speedup vs baseline: 1.7695x; 1.7695x over previous
import functools

import jax
import jax.numpy as jnp
from jax import lax
from jax.experimental import pallas as pl
from jax.experimental.pallas import tpu as pltpu

B = 16
R = 256
D = 1024
NOBJ = 64
DH1 = 1024
DH2 = 512
DO = 117


def _tc_body(nrel_ref, foo_ref, sp_ref, a_ref, g_ref,
             w1_ref, b1_ref, w2_ref, b2_ref, w3_ref, b3_ref, out_ref):
    b = pl.program_id(1)
    n = nrel_ref[b]
    xs = sp_ref[0]
    xr = foo_ref[0] * xs
    xp = jnp.dot(a_ref[0], g_ref[0], preferred_element_type=jnp.float32)
    mask = (lax.broadcasted_iota(jnp.int32, (R, DO), 0) < n).astype(jnp.float32)

    def classify(x, j):
        h = jnp.maximum(
            jnp.dot(x, w1_ref[j], preferred_element_type=jnp.float32) + b1_ref[0, j], 0.0)
        h = jnp.maximum(
            jnp.dot(h, w2_ref[j], preferred_element_type=jnp.float32) + b2_ref[0, j], 0.0)
        z = jnp.dot(h, w3_ref[j], preferred_element_type=jnp.float32) + b3_ref[0, j]
        return jax.nn.sigmoid(z)

    s = classify(xs, 0) * classify(xr, 1) * classify(xp, 2)
    out_ref[0] = s * mask


def kernel(f_oo_vis, spatial_branch_output, graphical_branch_output, obj_pairs,
           num_rels, W1, b1, W2, b2, W3, b3):
    p0 = obj_pairs[..., 0]
    p1 = obj_pairs[..., 1]
    onehot = (
        jax.nn.one_hot(p0, NOBJ, dtype=jnp.float32)
        + jax.nn.one_hot(p1, NOBJ, dtype=jnp.float32)
    ) * 0.5

    def reord(w):
        return w.reshape((3, 3) + w.shape[1:]).swapaxes(0, 1).reshape(w.shape)

    W1r, b1r = reord(W1), reord(b1).reshape(3, 3, DH1)
    W2r, b2r = reord(W2), reord(b2).reshape(3, 3, DH2)
    W3r, b3r = reord(W3), reord(b3).reshape(3, 3, DO)

    grid_spec = pltpu.PrefetchScalarGridSpec(
        num_scalar_prefetch=1,
        grid=(3, B),
        in_specs=[
            pl.BlockSpec((1, R, D), lambda k, b, nr: (b, 0, 0)),
            pl.BlockSpec((1, R, D), lambda k, b, nr: (b, 0, 0)),
            pl.BlockSpec((1, R, NOBJ), lambda k, b, nr: (b, 0, 0)),
            pl.BlockSpec((1, NOBJ, D), lambda k, b, nr: (b, 0, 0)),
            pl.BlockSpec((3, D, DH1), lambda k, b, nr: (k, 0, 0)),
            pl.BlockSpec((1, 3, DH1), lambda k, b, nr: (k, 0, 0)),
            pl.BlockSpec((3, DH1, DH2), lambda k, b, nr: (k, 0, 0)),
            pl.BlockSpec((1, 3, DH2), lambda k, b, nr: (k, 0, 0)),
            pl.BlockSpec((3, DH2, DO), lambda k, b, nr: (k, 0, 0)),
            pl.BlockSpec((1, 3, DO), lambda k, b, nr: (k, 0, 0)),
        ],
        out_specs=pl.BlockSpec((1, R, DO), lambda k, b, nr: (k, b, 0)),
    )
    out = pl.pallas_call(
        _tc_body,
        grid_spec=grid_spec,
        out_shape=jax.ShapeDtypeStruct((3, B * R, DO), jnp.float32),
    )(num_rels, f_oo_vis, spatial_branch_output, onehot, graphical_branch_output,
      W1r, b1r, W2r, b2r, W3r, b3r)
    return out


# speedup vs baseline: 2.0047x; 1.1329x over previous
import functools

import jax
import jax.numpy as jnp
from jax import lax
from jax.experimental import pallas as pl
from jax.experimental.pallas import tpu as pltpu

B = 16
R = 256
D = 1024
NOBJ = 64
DH1 = 1024
DH2 = 512
DO = 117


def _tc_body(nrel_ref, foo_ref, sp_ref, a_ref, g_ref,
             w1_ref, b1_ref, w2_ref, b2_ref, w3_ref, b3_ref, out_ref):
    b = pl.program_id(0)
    n = nrel_ref[b]
    xs = sp_ref[0]
    xr = foo_ref[0] * xs
    xp = jnp.dot(a_ref[0], g_ref[0],
                 preferred_element_type=jnp.float32).astype(jnp.bfloat16)
    mask = (lax.broadcasted_iota(jnp.int32, (R, DO), 0) < n).astype(jnp.float32)

    def classify(x, i):
        h = jnp.dot(x, w1_ref[i], preferred_element_type=jnp.float32) + b1_ref[i]
        h = jnp.maximum(h, 0.0).astype(jnp.bfloat16)
        h = jnp.dot(h, w2_ref[i], preferred_element_type=jnp.float32) + b2_ref[i]
        h = jnp.maximum(h, 0.0).astype(jnp.bfloat16)
        z = jnp.dot(h, w3_ref[i], preferred_element_type=jnp.float32) + b3_ref[i]
        return jax.nn.sigmoid(z)

    for k in range(3):
        s = classify(xs, k) * classify(xr, 3 + k) * classify(xp, 6 + k)
        out_ref[k] = s * mask


def kernel(f_oo_vis, spatial_branch_output, graphical_branch_output, obj_pairs,
           num_rels, W1, b1, W2, b2, W3, b3):
    p0 = obj_pairs[..., 0]
    p1 = obj_pairs[..., 1]
    onehot = (
        jax.nn.one_hot(p0, NOBJ, dtype=jnp.bfloat16)
        + jax.nn.one_hot(p1, NOBJ, dtype=jnp.bfloat16)
    ) * jnp.bfloat16(0.5)

    bf = jnp.bfloat16
    grid_spec = pltpu.PrefetchScalarGridSpec(
        num_scalar_prefetch=1,
        grid=(B,),
        in_specs=[
            pl.BlockSpec((1, R, D), lambda b, nr: (b, 0, 0)),
            pl.BlockSpec((1, R, D), lambda b, nr: (b, 0, 0)),
            pl.BlockSpec((1, R, NOBJ), lambda b, nr: (b, 0, 0)),
            pl.BlockSpec((1, NOBJ, D), lambda b, nr: (b, 0, 0)),
            pl.BlockSpec((9, D, DH1), lambda b, nr: (0, 0, 0)),
            pl.BlockSpec((9, DH1), lambda b, nr: (0, 0)),
            pl.BlockSpec((9, DH1, DH2), lambda b, nr: (0, 0, 0)),
            pl.BlockSpec((9, DH2), lambda b, nr: (0, 0)),
            pl.BlockSpec((9, DH2, DO), lambda b, nr: (0, 0, 0)),
            pl.BlockSpec((9, DO), lambda b, nr: (0, 0)),
        ],
        out_specs=pl.BlockSpec((3, R, DO), lambda b, nr: (0, b, 0)),
    )
    out = pl.pallas_call(
        _tc_body,
        grid_spec=grid_spec,
        out_shape=jax.ShapeDtypeStruct((3, B * R, DO), jnp.float32),
    )(num_rels, f_oo_vis.astype(bf), spatial_branch_output.astype(bf), onehot,
      graphical_branch_output.astype(bf),
      W1.astype(bf), b1, W2.astype(bf), b2, W3.astype(bf), b3)
    return out
